# Initial kernel scaffold; baseline (speedup 1.0000x reference)
import jax
import jax.numpy as jnp
from jax.experimental import pallas as pl


def kernel(x):
    raise NotImplementedError("write your pallas kernel here")


# speedup vs baseline: 19.9722x; 19.9722x over previous
import jax
import jax.numpy as jnp
from jax.experimental import pallas as pl

_K = 512


def _topk_mask_body(x_ref, o_ref):
    x = x_ref[...]
    bits = jax.lax.bitcast_convert_type(x, jnp.int32)
    ikey = jnp.where(bits < 0, bits ^ jnp.int32(0x7FFFFFFF), bits)

    int_min = jnp.int32(-(2**31))
    rows = x.shape[0]
    t = jnp.full((rows, 1), int_min, jnp.int32)
    cnt = jnp.sum((ikey >= 0).astype(jnp.int32), axis=1, keepdims=True)
    t = jnp.where(cnt >= _K, jnp.int32(0), t)
    for b in range(30, -1, -1):
        cand = t | jnp.int32(1 << b)
        cnt = jnp.sum((ikey >= cand).astype(jnp.int32), axis=1, keepdims=True)
        t = jnp.where(cnt >= _K, cand, t)

    gt = ikey > t
    eq = ikey == t
    cnt_gt = jnp.sum(gt.astype(jnp.int32), axis=1, keepdims=True)
    need = _K - cnt_gt
    col = jax.lax.broadcasted_iota(jnp.int32, x.shape, 1)
    jcut = jnp.zeros((rows, 1), jnp.int32)
    for b in range(12, -1, -1):
        cand = jcut | jnp.int32(1 << b)
        cnt = jnp.sum((eq & (col < cand)).astype(jnp.int32), axis=1,
                      keepdims=True)
        jcut = jnp.where(cnt < need, cand, jcut)
    keep = gt | (eq & (col <= jcut))
    o_ref[...] = jnp.where(keep, jnp.maximum(x, 0.0), 0.0)


def kernel(x):
    return pl.pallas_call(
        _topk_mask_body,
        out_shape=jax.ShapeDtypeStruct(x.shape, x.dtype),
    )(x)
